# baseline (speedup 1.0000x reference)
import functools

import jax
import jax.numpy as jnp
from jax import lax
from jax.experimental import pallas as pl
from jax.experimental.pallas import tpu as pltpu
from jax.experimental.pallas import tpu_sc as plsc

BATCH = 16384
N_FIELDS = 26
EMBED_DIM = 32
TOTAL = BATCH * N_FIELDS
NC, NS = 2, 16
NW = NC * NS
PER_W = TOTAL // NW
CHUNK = 128
N_CHUNKS = PER_W // CHUNK
VECS = CHUNK // 16
PERIOD_V = 13
NBUF = 8
ROUNDS = N_CHUNKS // NBUF


def _sc_gather(idx2d, off_pat, table):
    mesh = plsc.VectorSubcoreMesh(core_axis_name="c", subcore_axis_name="s")

    @functools.partial(
        pl.kernel,
        out_type=jax.ShapeDtypeStruct((TOTAL, EMBED_DIM), jnp.float32),
        mesh=mesh,
        compiler_params=pltpu.CompilerParams(use_tc_tiling_on_sc=False),
        scratch_types=[
            pltpu.VMEM((N_CHUNKS, CHUNK), jnp.int32),
            pltpu.VMEM((PERIOD_V, 16), jnp.int32),
            pltpu.VMEM((NBUF, CHUNK, EMBED_DIM), jnp.float32),
            pltpu.SemaphoreType.DMA((NBUF,)),
            pltpu.SemaphoreType.DMA((NBUF,)),
        ],
    )
    def k(idx_hbm, offs_hbm, table_hbm, out_hbm, idx_v, offs_v, rows_v,
          gsem, wsem):
        c = lax.axis_index("c")
        s = lax.axis_index("s")
        wid = s * NC + c
        base = wid * PER_W
        pltpu.sync_copy(idx_hbm.at[pl.ds(wid * N_CHUNKS, N_CHUNKS)], idx_v)
        pltpu.sync_copy(offs_hbm, offs_v)

        def add_offsets(j):
            for u in range(VECS):
                rp = lax.rem(j * VECS + u, PERIOD_V)
                idx_v[j, pl.ds(u * 16, 16)] = (
                    idx_v[j, pl.ds(u * 16, 16)] + offs_v[rp, :]
                )

        def gather_desc(j, b):
            return pltpu.make_async_copy(
                table_hbm.at[idx_v.at[j]], rows_v.at[b], gsem.at[b]
            )

        def write_desc(j, b):
            return pltpu.make_async_copy(
                rows_v.at[b],
                out_hbm.at[pl.ds(base + j * CHUNK, CHUNK)],
                wsem.at[b],
            )

        for b in range(NBUF):
            add_offsets(b)
        for b in range(NBUF):
            gather_desc(b, b).start()

        def round_body(r, _):
            @pl.when(r < ROUNDS - 1)
            def _offs():
                for b in range(NBUF):
                    add_offsets((r + 1) * NBUF + b)

            for b in range(NBUF):
                j = r * NBUF + b
                gather_desc(j, b).wait()
                write_desc(j, b).start()
            for b in range(NBUF):
                j = r * NBUF + b
                write_desc(j, b).wait()

                @pl.when(r < ROUNDS - 1)
                def _refill():
                    gather_desc((r + 1) * NBUF + b, b).start()

            return 0

        lax.fori_loop(0, ROUNDS, round_body, 0)

    return k(idx2d, off_pat, table)


TR_BLK = 16384


def _tc_transpose_table(embed_table):
    tbl_t = embed_table.T

    def body(in_ref, out_ref):
        ident = (
            lax.broadcasted_iota(jnp.int32, (EMBED_DIM, EMBED_DIM), 0)
            == lax.broadcasted_iota(jnp.int32, (EMBED_DIM, EMBED_DIM), 1)
        ).astype(jnp.float32)
        out_ref[...] = lax.dot_general(
            in_ref[...], ident, (((0,), (0,)), ((), ())),
            preferred_element_type=jnp.float32,
        )

    vocab = embed_table.shape[0]
    grid = (vocab + TR_BLK - 1) // TR_BLK
    return pl.pallas_call(
        body,
        grid=(grid,),
        in_specs=[pl.BlockSpec((EMBED_DIM, TR_BLK), lambda i: (0, i))],
        out_specs=pl.BlockSpec((TR_BLK, EMBED_DIM), lambda i: (i, 0)),
        out_shape=jax.ShapeDtypeStruct((vocab, EMBED_DIM), jnp.float32),
    )(tbl_t)


def kernel(sparse_indices, offsets, embed_table):
    table_rm = _tc_transpose_table(embed_table)
    idx2d = sparse_indices.reshape(TOTAL // CHUNK, CHUNK)
    off_pat = jnp.tile(offsets.reshape(N_FIELDS), PERIOD_V * 16 // N_FIELDS)
    off_pat = off_pat.reshape(PERIOD_V, 16)
    out = _sc_gather(idx2d, off_pat, table_rm)
    return out.reshape(BATCH, N_FIELDS, EMBED_DIM)


# speedup vs baseline: 1.6051x; 1.6051x over previous
import functools

import jax
import jax.numpy as jnp
from jax import lax
from jax.experimental import pallas as pl
from jax.experimental.pallas import tpu as pltpu
from jax.experimental.pallas import tpu_sc as plsc

BATCH = 16384
N_FIELDS = 26
EMBED_DIM = 32
TOTAL = BATCH * N_FIELDS
NC, NS = 2, 16
NW = NC * NS
PER_W = TOTAL // NW
CHUNK = 128
N_CHUNKS = PER_W // CHUNK
VECS = CHUNK // 16
PERIOD_V = 13
NBUF = 8
ROUNDS = N_CHUNKS // NBUF


def _sc_gather(idx2d, off_pat, table):
    mesh = plsc.VectorSubcoreMesh(core_axis_name="c", subcore_axis_name="s")

    @functools.partial(
        pl.kernel,
        out_type=jax.ShapeDtypeStruct((TOTAL, EMBED_DIM), jnp.float32),
        mesh=mesh,
        compiler_params=pltpu.CompilerParams(use_tc_tiling_on_sc=False),
        scratch_types=[
            pltpu.VMEM((N_CHUNKS, CHUNK), jnp.int32),
            pltpu.VMEM((PERIOD_V, 16), jnp.int32),
            pltpu.VMEM((NBUF, CHUNK, EMBED_DIM), jnp.float32),
            pltpu.SemaphoreType.DMA((NBUF,)),
            pltpu.SemaphoreType.DMA((NBUF,)),
        ],
    )
    def k(idx_hbm, offs_hbm, table_hbm, out_hbm, idx_v, offs_v, rows_v,
          gsem, wsem):
        c = lax.axis_index("c")
        s = lax.axis_index("s")
        wid = s * NC + c
        base = wid * PER_W
        pltpu.sync_copy(idx_hbm.at[pl.ds(wid * N_CHUNKS, N_CHUNKS)], idx_v)
        pltpu.sync_copy(offs_hbm, offs_v)

        def add_offsets(j):
            for u in range(VECS):
                rp = lax.rem(j * VECS + u, PERIOD_V)
                idx_v[j, pl.ds(u * 16, 16)] = (
                    idx_v[j, pl.ds(u * 16, 16)] + offs_v[rp, :]
                )

        def gather_desc(j, b):
            return pltpu.make_async_copy(
                table_hbm.at[idx_v.at[j]], rows_v.at[b], gsem.at[b]
            )

        def write_desc(j, b):
            return pltpu.make_async_copy(
                rows_v.at[b],
                out_hbm.at[pl.ds(base + j * CHUNK, CHUNK)],
                wsem.at[b],
            )

        for b in range(NBUF):
            add_offsets(b)
        for b in range(NBUF):
            gather_desc(b, b).start()

        def round_body(r, _):
            @pl.when(r < ROUNDS - 1)
            def _offs():
                for b in range(NBUF):
                    add_offsets((r + 1) * NBUF + b)

            for b in range(NBUF):
                j = r * NBUF + b
                gather_desc(j, b).wait()
                write_desc(j, b).start()
            for b in range(NBUF):
                j = r * NBUF + b
                write_desc(j, b).wait()

                @pl.when(r < ROUNDS - 1)
                def _refill():
                    gather_desc((r + 1) * NBUF + b, b).start()

            return 0

        lax.fori_loop(0, ROUNDS, round_body, 0)

    return k(idx2d, off_pat, table)


TR_BLK = 16384


def _tc_transpose_table(embed_table):
    tbl_t = embed_table.T

    def body(in_ref, out_ref, y_ref):
        ident = (
            lax.broadcasted_iota(jnp.int32, (EMBED_DIM, EMBED_DIM), 0)
            == lax.broadcasted_iota(jnp.int32, (EMBED_DIM, EMBED_DIM), 1)
        ).astype(jnp.float32)
        y_ref[...] = lax.dot_general(
            in_ref[...], ident, (((0,), (0,)), ((), ())),
            preferred_element_type=jnp.float32,
        )
        for k in range(4):
            out_ref[:, k * EMBED_DIM:(k + 1) * EMBED_DIM] = (
                y_ref[pl.ds(k, TR_BLK // 4, 4), :]
            )

    vocab = embed_table.shape[0]
    grid = (vocab + TR_BLK - 1) // TR_BLK
    packed = pl.pallas_call(
        body,
        grid=(grid,),
        in_specs=[pl.BlockSpec((EMBED_DIM, TR_BLK), lambda i: (0, i))],
        out_specs=pl.BlockSpec((TR_BLK // 4, 4 * EMBED_DIM), lambda i: (i, 0)),
        out_shape=jax.ShapeDtypeStruct((vocab // 4, 4 * EMBED_DIM), jnp.float32),
        scratch_shapes=[pltpu.VMEM((TR_BLK, EMBED_DIM), jnp.float32)],
    )(tbl_t)
    return packed.reshape(vocab, EMBED_DIM)


def kernel(sparse_indices, offsets, embed_table):
    table_rm = _tc_transpose_table(embed_table)
    idx2d = sparse_indices.reshape(TOTAL // CHUNK, CHUNK)
    off_pat = jnp.tile(offsets.reshape(N_FIELDS), PERIOD_V * 16 // N_FIELDS)
    off_pat = off_pat.reshape(PERIOD_V, 16)
    out = _sc_gather(idx2d, off_pat, table_rm)
    return out.reshape(BATCH, N_FIELDS, EMBED_DIM)


# speedup vs baseline: 1.8206x; 1.1343x over previous
import functools

import jax
import jax.numpy as jnp
from jax import lax
from jax.experimental import pallas as pl
from jax.experimental.pallas import tpu as pltpu
from jax.experimental.pallas import tpu_sc as plsc

BATCH = 16384
N_FIELDS = 26
EMBED_DIM = 32
TOTAL = BATCH * N_FIELDS
NC, NS = 2, 16
NW = NC * NS
PER_W = TOTAL // NW
CHUNK = 128
N_CHUNKS = PER_W // CHUNK
VECS = CHUNK // 16
PERIOD_V = 13
NBUF = 8
ROUNDS = N_CHUNKS // NBUF


def _sc_gather(idx2d, off_pat, table):
    mesh = plsc.VectorSubcoreMesh(core_axis_name="c", subcore_axis_name="s")

    @functools.partial(
        pl.kernel,
        out_type=jax.ShapeDtypeStruct((TOTAL, EMBED_DIM), jnp.float32),
        mesh=mesh,
        compiler_params=pltpu.CompilerParams(use_tc_tiling_on_sc=False),
        scratch_types=[
            pltpu.VMEM((N_CHUNKS, CHUNK), jnp.int32),
            pltpu.VMEM((PERIOD_V, 16), jnp.int32),
            pltpu.VMEM((NBUF, CHUNK, EMBED_DIM), jnp.float32),
            pltpu.SemaphoreType.DMA((NBUF,)),
            pltpu.SemaphoreType.DMA((NBUF,)),
        ],
    )
    def k(idx_hbm, offs_hbm, table_hbm, out_hbm, idx_v, offs_v, rows_v,
          gsem, wsem):
        c = lax.axis_index("c")
        s = lax.axis_index("s")
        wid = s * NC + c
        base = wid * PER_W
        pltpu.sync_copy(idx_hbm.at[pl.ds(wid * N_CHUNKS, N_CHUNKS)], idx_v)
        pltpu.sync_copy(offs_hbm, offs_v)

        def add_offsets(j):
            for u in range(VECS):
                rp = lax.rem(j * VECS + u, PERIOD_V)
                idx_v[j, pl.ds(u * 16, 16)] = (
                    idx_v[j, pl.ds(u * 16, 16)] + offs_v[rp, :]
                )

        def gather_desc(j, b):
            return pltpu.make_async_copy(
                table_hbm.at[idx_v.at[j]], rows_v.at[b], gsem.at[b]
            )

        def write_desc(j, b):
            return pltpu.make_async_copy(
                rows_v.at[b],
                out_hbm.at[pl.ds(base + j * CHUNK, CHUNK)],
                wsem.at[b],
            )

        for b in range(NBUF):
            add_offsets(b)
        for b in range(NBUF):
            gather_desc(b, b).start()

        def round_body(r, _):
            @pl.when(r < ROUNDS - 1)
            def _offs():
                for b in range(NBUF):
                    add_offsets((r + 1) * NBUF + b)

            for b in range(NBUF):
                j = r * NBUF + b
                gather_desc(j, b).wait()
                write_desc(j, b).start()
            for b in range(NBUF):
                j = r * NBUF + b
                write_desc(j, b).wait()

                @pl.when(r < ROUNDS - 1)
                def _refill():
                    gather_desc((r + 1) * NBUF + b, b).start()

            return 0

        lax.fori_loop(0, ROUNDS, round_body, 0)

    return k(idx2d, off_pat, table)


TR_BLK = 16384


def _tc_transpose_table(embed_table):
    tbl_t = embed_table.T

    def body(in_ref, out_ref, y_ref):
        ident = (
            lax.broadcasted_iota(jnp.int32, (EMBED_DIM, EMBED_DIM), 0)
            == lax.broadcasted_iota(jnp.int32, (EMBED_DIM, EMBED_DIM), 1)
        ).astype(jnp.float32)
        y_ref[...] = lax.dot_general(
            in_ref[...], ident, (((0,), (0,)), ((), ())),
            preferred_element_type=jnp.float32,
        )
        out_ref[...] = jnp.concatenate(
            [y_ref[pl.ds(k, TR_BLK // 4, 4), :] for k in range(4)], axis=1
        )

    vocab = embed_table.shape[0]
    grid = (vocab + TR_BLK - 1) // TR_BLK
    packed = pl.pallas_call(
        body,
        grid=(grid,),
        in_specs=[pl.BlockSpec((EMBED_DIM, TR_BLK), lambda i: (0, i))],
        out_specs=pl.BlockSpec((TR_BLK // 4, 4 * EMBED_DIM), lambda i: (i, 0)),
        out_shape=jax.ShapeDtypeStruct((vocab // 4, 4 * EMBED_DIM), jnp.float32),
        scratch_shapes=[pltpu.VMEM((TR_BLK, EMBED_DIM), jnp.float32)],
    )(tbl_t)
    return packed.reshape(vocab, EMBED_DIM)


def kernel(sparse_indices, offsets, embed_table):
    table_rm = _tc_transpose_table(embed_table)
    idx2d = sparse_indices.reshape(TOTAL // CHUNK, CHUNK)
    off_pat = jnp.tile(offsets.reshape(N_FIELDS), PERIOD_V * 16 // N_FIELDS)
    off_pat = off_pat.reshape(PERIOD_V, 16)
    out = _sc_gather(idx2d, off_pat, table_rm)
    return out.reshape(BATCH, N_FIELDS, EMBED_DIM)


# speedup vs baseline: 1.8328x; 1.0067x over previous
import functools

import jax
import jax.numpy as jnp
from jax import lax
from jax.experimental import pallas as pl
from jax.experimental.pallas import tpu as pltpu
from jax.experimental.pallas import tpu_sc as plsc

BATCH = 16384
N_FIELDS = 26
EMBED_DIM = 32
TOTAL = BATCH * N_FIELDS
NC, NS = 2, 16
NW = NC * NS
PER_W = TOTAL // NW
CHUNK = 128
N_CHUNKS = PER_W // CHUNK
VECS = CHUNK // 16
PERIOD_V = 13
NBUF = 8
ROUNDS = N_CHUNKS // NBUF


def _sc_gather(idx2d, off_pat, table):
    mesh = plsc.VectorSubcoreMesh(core_axis_name="c", subcore_axis_name="s")

    @functools.partial(
        pl.kernel,
        out_type=jax.ShapeDtypeStruct((TOTAL, EMBED_DIM), jnp.float32),
        mesh=mesh,
        compiler_params=pltpu.CompilerParams(use_tc_tiling_on_sc=False),
        scratch_types=[
            pltpu.VMEM((N_CHUNKS, CHUNK), jnp.int32),
            pltpu.VMEM((PERIOD_V, 16), jnp.int32),
            pltpu.VMEM((NBUF, CHUNK, EMBED_DIM), jnp.float32),
            pltpu.SemaphoreType.DMA((NBUF,)),
            pltpu.SemaphoreType.DMA((NBUF,)),
        ],
    )
    def k(idx_hbm, offs_hbm, table_hbm, out_hbm, idx_v, offs_v, rows_v,
          gsem, wsem):
        c = lax.axis_index("c")
        s = lax.axis_index("s")
        wid = s * NC + c
        base = wid * PER_W
        pltpu.sync_copy(idx_hbm.at[pl.ds(wid * N_CHUNKS, N_CHUNKS)], idx_v)
        pltpu.sync_copy(offs_hbm, offs_v)

        def add_offsets(j):
            for u in range(VECS):
                rp = lax.rem(j * VECS + u, PERIOD_V)
                idx_v[j, pl.ds(u * 16, 16)] = (
                    idx_v[j, pl.ds(u * 16, 16)] + offs_v[rp, :]
                )

        def gather_desc(j, b):
            return pltpu.make_async_copy(
                table_hbm.at[idx_v.at[j]], rows_v.at[b], gsem.at[b]
            )

        def write_desc(j, b):
            return pltpu.make_async_copy(
                rows_v.at[b],
                out_hbm.at[pl.ds(base + j * CHUNK, CHUNK)],
                wsem.at[b],
            )

        for b in range(NBUF):
            add_offsets(b)
        for b in range(NBUF):
            gather_desc(b, b).start()

        def round_body(r, _):
            @pl.when(r < ROUNDS - 1)
            def _offs():
                for b in range(NBUF):
                    add_offsets((r + 1) * NBUF + b)

            for b in range(NBUF):
                j = r * NBUF + b
                gather_desc(j, b).wait()
                write_desc(j, b).start()
            for b in range(NBUF):
                j = r * NBUF + b
                write_desc(j, b).wait()

                @pl.when(r < ROUNDS - 1)
                def _refill():
                    gather_desc((r + 1) * NBUF + b, b).start()

            return 0

        lax.fori_loop(0, ROUNDS, round_body, 0)

    return k(idx2d, off_pat, table)


TR_BLK = 32768


def _tc_transpose_table(embed_table):
    tbl_t = embed_table.T

    def body(in_ref, out_ref, y_ref):
        y_ref[...] = in_ref[...].T
        out_ref[...] = jnp.concatenate(
            [y_ref[pl.ds(k, TR_BLK // 4, 4), :] for k in range(4)], axis=1
        )

    vocab = embed_table.shape[0]
    grid = (vocab + TR_BLK - 1) // TR_BLK
    packed = pl.pallas_call(
        body,
        grid=(grid,),
        in_specs=[pl.BlockSpec((EMBED_DIM, TR_BLK), lambda i: (0, i))],
        out_specs=pl.BlockSpec((TR_BLK // 4, 4 * EMBED_DIM), lambda i: (i, 0)),
        out_shape=jax.ShapeDtypeStruct((vocab // 4, 4 * EMBED_DIM), jnp.float32),
        scratch_shapes=[pltpu.VMEM((TR_BLK, EMBED_DIM), jnp.float32)],
    )(tbl_t)
    return packed.reshape(vocab, EMBED_DIM)


def kernel(sparse_indices, offsets, embed_table):
    table_rm = _tc_transpose_table(embed_table)
    idx2d = sparse_indices.reshape(TOTAL // CHUNK, CHUNK)
    off_pat = jnp.tile(offsets.reshape(N_FIELDS), PERIOD_V * 16 // N_FIELDS)
    off_pat = off_pat.reshape(PERIOD_V, 16)
    out = _sc_gather(idx2d, off_pat, table_rm)
    return out.reshape(BATCH, N_FIELDS, EMBED_DIM)


# speedup vs baseline: 3.0028x; 1.6384x over previous
import functools

import jax
import jax.numpy as jnp
from jax import lax
from jax.experimental import pallas as pl
from jax.experimental.pallas import tpu as pltpu
from jax.experimental.pallas import tpu_sc as plsc

BATCH = 16384
N_FIELDS = 26
EMBED_DIM = 32
TOTAL = BATCH * N_FIELDS
NC, NS = 2, 16
NW = NC * NS
PER_W = TOTAL // NW
CHUNK = 128
N_CHUNKS = PER_W // CHUNK
VECS = CHUNK // 16
PERIOD_V = 13
NBUF = 8
ROUNDS = N_CHUNKS // NBUF


def _sc_gather(idx2d, off_pat, table):
    mesh = plsc.VectorSubcoreMesh(core_axis_name="c", subcore_axis_name="s")

    @functools.partial(
        pl.kernel,
        out_type=jax.ShapeDtypeStruct((TOTAL, EMBED_DIM), jnp.float32),
        mesh=mesh,
        compiler_params=pltpu.CompilerParams(use_tc_tiling_on_sc=False),
        scratch_types=[
            pltpu.VMEM((N_CHUNKS, CHUNK), jnp.int32),
            pltpu.VMEM((PERIOD_V, 16), jnp.int32),
            pltpu.VMEM((NBUF, CHUNK, EMBED_DIM), jnp.float32),
            pltpu.SemaphoreType.DMA((NBUF,)),
            pltpu.SemaphoreType.DMA((NBUF,)),
        ],
    )
    def k(idx_hbm, offs_hbm, table_hbm, out_hbm, idx_v, offs_v, rows_v,
          gsem, wsem):
        c = lax.axis_index("c")
        s = lax.axis_index("s")
        wid = s * NC + c
        base = wid * PER_W
        pltpu.sync_copy(idx_hbm.at[pl.ds(wid * N_CHUNKS, N_CHUNKS)], idx_v)
        pltpu.sync_copy(offs_hbm, offs_v)

        def add_offsets(j):
            for u in range(VECS):
                rp = lax.rem(j * VECS + u, PERIOD_V)
                v = idx_v[j, pl.ds(u * 16, 16)] + offs_v[rp, :]
                idx_v[j, pl.ds(u * 16, 16)] = (
                    (v & (-TR_BLK))
                    + ((v & (TR_SUB - 1)) << 2)
                    + ((v >> 12) & 3)
                )

        def gather_desc(j, b):
            return pltpu.make_async_copy(
                table_hbm.at[idx_v.at[j]], rows_v.at[b], gsem.at[b]
            )

        def write_desc(j, b):
            return pltpu.make_async_copy(
                rows_v.at[b],
                out_hbm.at[pl.ds(base + j * CHUNK, CHUNK)],
                wsem.at[b],
            )

        for b in range(NBUF):
            add_offsets(b)
        for b in range(NBUF):
            gather_desc(b, b).start()

        def round_body(r, _):
            @pl.when(r < ROUNDS - 1)
            def _offs():
                for b in range(NBUF):
                    add_offsets((r + 1) * NBUF + b)

            for b in range(NBUF):
                j = r * NBUF + b
                gather_desc(j, b).wait()
                write_desc(j, b).start()
            for b in range(NBUF):
                j = r * NBUF + b
                write_desc(j, b).wait()

                @pl.when(r < ROUNDS - 1)
                def _refill():
                    gather_desc((r + 1) * NBUF + b, b).start()

            return 0

        lax.fori_loop(0, ROUNDS, round_body, 0)

    return k(idx2d, off_pat, table)


TR_BLK = 16384
TR_SUB = TR_BLK // 4


def _tc_transpose_table(embed_table):
    tbl_t = embed_table.T

    def body(in_ref, out_ref):
        x = in_ref[...]
        x4 = jnp.concatenate(
            [x[:, a * TR_SUB:(a + 1) * TR_SUB] for a in range(4)], axis=0
        )
        out_ref[...] = x4.T

    vocab = embed_table.shape[0]
    grid = (vocab + TR_BLK - 1) // TR_BLK
    packed = pl.pallas_call(
        body,
        grid=(grid,),
        in_specs=[pl.BlockSpec((EMBED_DIM, TR_BLK), lambda i: (0, i))],
        out_specs=pl.BlockSpec((TR_SUB, 4 * EMBED_DIM), lambda i: (i, 0)),
        out_shape=jax.ShapeDtypeStruct((grid * TR_SUB, 4 * EMBED_DIM),
                                       jnp.float32),
    )(tbl_t)
    return packed.reshape(grid * TR_BLK, EMBED_DIM)


def kernel(sparse_indices, offsets, embed_table):
    table_rm = _tc_transpose_table(embed_table)
    idx2d = sparse_indices.reshape(TOTAL // CHUNK, CHUNK)
    off_pat = jnp.tile(offsets.reshape(N_FIELDS), PERIOD_V * 16 // N_FIELDS)
    off_pat = off_pat.reshape(PERIOD_V, 16)
    out = _sc_gather(idx2d, off_pat, table_rm)
    return out.reshape(BATCH, N_FIELDS, EMBED_DIM)


# speedup vs baseline: 4.7813x; 1.5923x over previous
import functools

import jax
import jax.numpy as jnp
from jax import lax
from jax.experimental import pallas as pl
from jax.experimental.pallas import tpu as pltpu
from jax.experimental.pallas import tpu_sc as plsc

BATCH = 16384
N_FIELDS = 26
EMBED_DIM = 32
TOTAL = BATCH * N_FIELDS
NC, NS = 2, 16
NW = NC * NS
PER_W = TOTAL // NW
CHUNK = 128
N_CHUNKS = PER_W // CHUNK
VECS = CHUNK // 16
NBUF = 8
ROUNDS = N_CHUNKS // NBUF
CPF = BATCH // CHUNK

TR_BLK = 16384
TR_SUB = TR_BLK // 4
OUT_SUB = BATCH // 4


def _sc_gather(idx2d, off_pat, table):
    mesh = plsc.VectorSubcoreMesh(core_axis_name="c", subcore_axis_name="s")

    @functools.partial(
        pl.kernel,
        out_type=jax.ShapeDtypeStruct((TOTAL // 4, 4 * EMBED_DIM), jnp.float32),
        mesh=mesh,
        compiler_params=pltpu.CompilerParams(use_tc_tiling_on_sc=False),
        scratch_types=[
            pltpu.VMEM((N_CHUNKS, CHUNK), jnp.int32),
            pltpu.VMEM((N_FIELDS, 16), jnp.int32),
            pltpu.VMEM((NBUF, CHUNK, EMBED_DIM), jnp.float32),
            pltpu.SemaphoreType.DMA((NBUF,)),
            pltpu.SemaphoreType.DMA((NBUF,)),
        ],
    )
    def k(idx_hbm, offs_hbm, table_hbm, out_hbm, idx_v, offs_v, rows_v,
          gsem, wsem):
        c = lax.axis_index("c")
        s = lax.axis_index("s")
        wid = s * NC + c
        gbase = wid * N_CHUNKS
        pltpu.sync_copy(idx_hbm.at[pl.ds(gbase, N_CHUNKS)], idx_v)
        pltpu.sync_copy(offs_hbm, offs_v)

        def add_offsets(j):
            f = (gbase + j) // CPF
            off = offs_v[f, :]
            for u in range(VECS):
                v = idx_v[j, pl.ds(u * 16, 16)] + off
                idx_v[j, pl.ds(u * 16, 16)] = (
                    (v & (-TR_BLK))
                    + ((v & (TR_SUB - 1)) << 2)
                    + ((v >> 12) & 3)
                )

        def gather_desc(j, b):
            return pltpu.make_async_copy(
                table_hbm.at[idx_v.at[j]], rows_v.at[b], gsem.at[b]
            )

        def write_desc(j, b):
            g = gbase + j
            f = g // CPF
            b0 = (g % CPF) * CHUNK
            c0 = f * OUT_SUB + (b0 & (OUT_SUB - 1))
            a0 = b0 >> 12
            return pltpu.make_async_copy(
                rows_v.at[b],
                out_hbm.at[pl.ds(c0, CHUNK),
                           pl.ds(a0 * EMBED_DIM, EMBED_DIM)],
                wsem.at[b],
            )

        for b in range(NBUF):
            add_offsets(b)
        for b in range(NBUF):
            gather_desc(b, b).start()

        def round_body(r, _):
            @pl.when(r < ROUNDS - 1)
            def _offs():
                for b in range(NBUF):
                    add_offsets((r + 1) * NBUF + b)

            for b in range(NBUF):
                j = r * NBUF + b
                gather_desc(j, b).wait()
                write_desc(j, b).start()
            for b in range(NBUF):
                j = r * NBUF + b
                write_desc(j, b).wait()

                @pl.when(r < ROUNDS - 1)
                def _refill():
                    gather_desc((r + 1) * NBUF + b, b).start()

            return 0

        lax.fori_loop(0, ROUNDS, round_body, 0)

    return k(idx2d, off_pat, table)


def _tc_transpose_table(embed_table):
    tbl_t = embed_table.T

    def body(in_ref, out_ref):
        x = in_ref[...]
        x4 = jnp.concatenate(
            [x[:, a * TR_SUB:(a + 1) * TR_SUB] for a in range(4)], axis=0
        )
        out_ref[...] = x4.T

    vocab = embed_table.shape[0]
    grid = (vocab + TR_BLK - 1) // TR_BLK
    packed = pl.pallas_call(
        body,
        grid=(grid,),
        in_specs=[pl.BlockSpec((EMBED_DIM, TR_BLK), lambda i: (0, i))],
        out_specs=pl.BlockSpec((TR_SUB, 4 * EMBED_DIM), lambda i: (i, 0)),
        out_shape=jax.ShapeDtypeStruct((grid * TR_SUB, 4 * EMBED_DIM),
                                       jnp.float32),
    )(tbl_t)
    return packed.reshape(grid * TR_BLK, EMBED_DIM)


def _tc_format_output(gp):
    def body(in_ref, out_ref):
        y = in_ref[...].T
        out_ref[0] = jnp.concatenate(
            [y[a * EMBED_DIM:(a + 1) * EMBED_DIM, :] for a in range(4)],
            axis=1,
        )

    return pl.pallas_call(
        body,
        grid=(N_FIELDS,),
        in_specs=[pl.BlockSpec((OUT_SUB, 4 * EMBED_DIM), lambda f: (f, 0))],
        out_specs=pl.BlockSpec((1, EMBED_DIM, BATCH), lambda f: (f, 0, 0)),
        out_shape=jax.ShapeDtypeStruct((N_FIELDS, EMBED_DIM, BATCH),
                                       jnp.float32),
    )(gp)


def kernel(sparse_indices, offsets, embed_table):
    table_rm = _tc_transpose_table(embed_table)
    idx2d = sparse_indices.T.reshape(TOTAL // CHUNK, CHUNK)
    off_pat = jnp.tile(offsets.reshape(N_FIELDS, 1), (1, 16))
    gp = _sc_gather(idx2d, off_pat, table_rm)
    out_t = _tc_format_output(gp)
    return jnp.transpose(out_t, (2, 0, 1))


# speedup vs baseline: 5.3449x; 1.1179x over previous
import functools

import jax
import jax.numpy as jnp
from jax import lax
from jax.experimental import pallas as pl
from jax.experimental.pallas import tpu as pltpu
from jax.experimental.pallas import tpu_sc as plsc

BATCH = 16384
N_FIELDS = 26
EMBED_DIM = 32
TOTAL = BATCH * N_FIELDS
NC, NS = 2, 16
NW = NC * NS
PER_W = TOTAL // NW
CHUNK = 128
N_CHUNKS = PER_W // CHUNK
VECS = CHUNK // 16
NBUF = 8
ROUNDS = N_CHUNKS // NBUF
CPF = BATCH // CHUNK

TR_BLK = 32768
TR_SUB = TR_BLK // 4
TR_SHIFT = TR_SUB.bit_length() - 1
OUT_SUB = BATCH // 4


def _sc_gather(idx2d, off_pat, table):
    mesh = plsc.VectorSubcoreMesh(core_axis_name="c", subcore_axis_name="s")

    @functools.partial(
        pl.kernel,
        out_type=jax.ShapeDtypeStruct((TOTAL // 4, 4 * EMBED_DIM), jnp.float32),
        mesh=mesh,
        compiler_params=pltpu.CompilerParams(use_tc_tiling_on_sc=False),
        scratch_types=[
            pltpu.VMEM((N_CHUNKS, CHUNK), jnp.int32),
            pltpu.VMEM((N_FIELDS, 16), jnp.int32),
            pltpu.VMEM((NBUF, CHUNK, EMBED_DIM), jnp.float32),
            pltpu.SemaphoreType.DMA((NBUF,)),
            pltpu.SemaphoreType.DMA((NBUF,)),
        ],
    )
    def k(idx_hbm, offs_hbm, table_hbm, out_hbm, idx_v, offs_v, rows_v,
          gsem, wsem):
        c = lax.axis_index("c")
        s = lax.axis_index("s")
        wid = s * NC + c
        gbase = wid * N_CHUNKS
        pltpu.sync_copy(idx_hbm.at[pl.ds(gbase, N_CHUNKS)], idx_v)
        pltpu.sync_copy(offs_hbm, offs_v)

        def add_offsets(j):
            f = (gbase + j) // CPF
            off = offs_v[f, :]
            for u in range(VECS):
                v = idx_v[j, pl.ds(u * 16, 16)] + off
                idx_v[j, pl.ds(u * 16, 16)] = (
                    (v & (-TR_BLK))
                    + ((v & (TR_SUB - 1)) << 2)
                    + ((v >> TR_SHIFT) & 3)
                )

        def gather_desc(j, b):
            return pltpu.make_async_copy(
                table_hbm.at[idx_v.at[j]], rows_v.at[b], gsem.at[b]
            )

        def write_desc(j, b):
            g = gbase + j
            f = g // CPF
            b0 = (g % CPF) * CHUNK
            c0 = f * OUT_SUB + (b0 & (OUT_SUB - 1))
            a0 = b0 >> 12
            return pltpu.make_async_copy(
                rows_v.at[b],
                out_hbm.at[pl.ds(c0, CHUNK),
                           pl.ds(a0 * EMBED_DIM, EMBED_DIM)],
                wsem.at[b],
            )

        for b in range(NBUF):
            add_offsets(b)
        for b in range(NBUF):
            gather_desc(b, b).start()

        def round_body(r, _):
            @pl.when(r < ROUNDS - 1)
            def _offs():
                for b in range(NBUF):
                    add_offsets((r + 1) * NBUF + b)

            for b in range(NBUF):
                j = r * NBUF + b
                gather_desc(j, b).wait()
                write_desc(j, b).start()
            for b in range(NBUF):
                j = r * NBUF + b
                write_desc(j, b).wait()

                @pl.when(r < ROUNDS - 1)
                def _refill():
                    gather_desc((r + 1) * NBUF + b, b).start()

            return 0

        lax.fori_loop(0, ROUNDS, round_body, 0)

    return k(idx2d, off_pat, table)


def _tc_transpose_table(embed_table):
    tbl_t = embed_table.T

    def body(in_ref, out_ref):
        x = in_ref[...]
        x4 = jnp.concatenate(
            [x[:, a * TR_SUB:(a + 1) * TR_SUB] for a in range(4)], axis=0
        )
        out_ref[...] = x4.T

    vocab = embed_table.shape[0]
    grid = (vocab + TR_BLK - 1) // TR_BLK
    packed = pl.pallas_call(
        body,
        grid=(grid,),
        in_specs=[pl.BlockSpec((EMBED_DIM, TR_BLK), lambda i: (0, i))],
        out_specs=pl.BlockSpec((TR_SUB, 4 * EMBED_DIM), lambda i: (i, 0)),
        out_shape=jax.ShapeDtypeStruct((grid * TR_SUB, 4 * EMBED_DIM),
                                       jnp.float32),
    )(tbl_t)
    return packed.reshape(grid * TR_BLK, EMBED_DIM)


def _tc_format_output(gp):
    def body(in_ref, out_ref):
        y = in_ref[...].T
        out_ref[0] = jnp.concatenate(
            [y[a * EMBED_DIM:(a + 1) * EMBED_DIM, :] for a in range(4)],
            axis=1,
        )

    return pl.pallas_call(
        body,
        grid=(N_FIELDS,),
        in_specs=[pl.BlockSpec((OUT_SUB, 4 * EMBED_DIM), lambda f: (f, 0))],
        out_specs=pl.BlockSpec((1, EMBED_DIM, BATCH), lambda f: (f, 0, 0)),
        out_shape=jax.ShapeDtypeStruct((N_FIELDS, EMBED_DIM, BATCH),
                                       jnp.float32),
    )(gp)


def kernel(sparse_indices, offsets, embed_table):
    table_rm = _tc_transpose_table(embed_table)
    idx2d = sparse_indices.T.reshape(TOTAL // CHUNK, CHUNK)
    off_pat = jnp.tile(offsets.reshape(N_FIELDS, 1), (1, 16))
    gp = _sc_gather(idx2d, off_pat, table_rm)
    out_t = _tc_format_output(gp)
    return jnp.transpose(out_t, (2, 0, 1))


# speedup vs baseline: 5.4136x; 1.0128x over previous
import functools

import jax
import jax.numpy as jnp
from jax import lax
from jax.experimental import pallas as pl
from jax.experimental.pallas import tpu as pltpu
from jax.experimental.pallas import tpu_sc as plsc

BATCH = 16384
N_FIELDS = 26
EMBED_DIM = 32
TOTAL = BATCH * N_FIELDS
NC, NS = 2, 16
NW = NC * NS
PER_W = TOTAL // NW
CHUNK = 128
N_CHUNKS = PER_W // CHUNK
VECS = CHUNK // 16
NBUF = 8
ROUNDS = N_CHUNKS // NBUF
CPF = BATCH // CHUNK

TR_BLK = 65536
TR_SUB = TR_BLK // 4
TR_SHIFT = TR_SUB.bit_length() - 1
OUT_SUB = BATCH // 4


def _sc_gather(idx2d, off_pat, table):
    mesh = plsc.VectorSubcoreMesh(core_axis_name="c", subcore_axis_name="s")

    @functools.partial(
        pl.kernel,
        out_type=jax.ShapeDtypeStruct((TOTAL // 4, 4 * EMBED_DIM), jnp.float32),
        mesh=mesh,
        compiler_params=pltpu.CompilerParams(use_tc_tiling_on_sc=False),
        scratch_types=[
            pltpu.VMEM((N_CHUNKS, CHUNK), jnp.int32),
            pltpu.VMEM((N_FIELDS, 16), jnp.int32),
            pltpu.VMEM((NBUF, CHUNK, EMBED_DIM), jnp.float32),
            pltpu.SemaphoreType.DMA((NBUF,)),
            pltpu.SemaphoreType.DMA((NBUF,)),
        ],
    )
    def k(idx_hbm, offs_hbm, table_hbm, out_hbm, idx_v, offs_v, rows_v,
          gsem, wsem):
        c = lax.axis_index("c")
        s = lax.axis_index("s")
        wid = s * NC + c
        gbase = wid * N_CHUNKS
        pltpu.sync_copy(idx_hbm.at[pl.ds(gbase, N_CHUNKS)], idx_v)
        pltpu.sync_copy(offs_hbm, offs_v)

        def add_offsets(j):
            f = (gbase + j) // CPF
            off = offs_v[f, :]
            for u in range(VECS):
                v = idx_v[j, pl.ds(u * 16, 16)] + off
                idx_v[j, pl.ds(u * 16, 16)] = (
                    (v & (-TR_BLK))
                    + ((v & (TR_SUB - 1)) << 2)
                    + ((v >> TR_SHIFT) & 3)
                )

        def gather_desc(j, b):
            return pltpu.make_async_copy(
                table_hbm.at[idx_v.at[j]], rows_v.at[b], gsem.at[b]
            )

        def write_desc(j, b):
            g = gbase + j
            f = g // CPF
            b0 = (g % CPF) * CHUNK
            c0 = f * OUT_SUB + (b0 & (OUT_SUB - 1))
            a0 = b0 >> 12
            return pltpu.make_async_copy(
                rows_v.at[b],
                out_hbm.at[pl.ds(c0, CHUNK),
                           pl.ds(a0 * EMBED_DIM, EMBED_DIM)],
                wsem.at[b],
            )

        for b in range(NBUF):
            add_offsets(b)
        for b in range(NBUF):
            gather_desc(b, b).start()

        def round_body(r, _):
            @pl.when(r < ROUNDS - 1)
            def _offs():
                for b in range(NBUF):
                    add_offsets((r + 1) * NBUF + b)

            for b in range(NBUF):
                j = r * NBUF + b
                gather_desc(j, b).wait()
                write_desc(j, b).start()
            for b in range(NBUF):
                j = r * NBUF + b
                write_desc(j, b).wait()

                @pl.when(r < ROUNDS - 1)
                def _refill():
                    gather_desc((r + 1) * NBUF + b, b).start()

            return 0

        lax.fori_loop(0, ROUNDS, round_body, 0)

    return k(idx2d, off_pat, table)


def _tc_transpose_table(embed_table):
    tbl_t = embed_table.T

    def body(in_ref, out_ref):
        x = in_ref[...]
        x4 = jnp.concatenate(
            [x[:, a * TR_SUB:(a + 1) * TR_SUB] for a in range(4)], axis=0
        )
        out_ref[...] = x4.T

    vocab = embed_table.shape[0]
    grid = (vocab + TR_BLK - 1) // TR_BLK
    packed = pl.pallas_call(
        body,
        grid=(grid,),
        in_specs=[pl.BlockSpec((EMBED_DIM, TR_BLK), lambda i: (0, i))],
        out_specs=pl.BlockSpec((TR_SUB, 4 * EMBED_DIM), lambda i: (i, 0)),
        out_shape=jax.ShapeDtypeStruct((grid * TR_SUB, 4 * EMBED_DIM),
                                       jnp.float32),
    )(tbl_t)
    return packed.reshape(grid * TR_BLK, EMBED_DIM)


def _tc_format_output(gp):
    def body(in_ref, out_ref):
        y = in_ref[...].T
        out_ref[0] = jnp.concatenate(
            [y[a * EMBED_DIM:(a + 1) * EMBED_DIM, :] for a in range(4)],
            axis=1,
        )

    return pl.pallas_call(
        body,
        grid=(N_FIELDS,),
        in_specs=[pl.BlockSpec((OUT_SUB, 4 * EMBED_DIM), lambda f: (f, 0))],
        out_specs=pl.BlockSpec((1, EMBED_DIM, BATCH), lambda f: (f, 0, 0)),
        out_shape=jax.ShapeDtypeStruct((N_FIELDS, EMBED_DIM, BATCH),
                                       jnp.float32),
    )(gp)


def kernel(sparse_indices, offsets, embed_table):
    table_rm = _tc_transpose_table(embed_table)
    idx2d = sparse_indices.T.reshape(TOTAL // CHUNK, CHUNK)
    off_pat = jnp.tile(offsets.reshape(N_FIELDS, 1), (1, 16))
    gp = _sc_gather(idx2d, off_pat, table_rm)
    out_t = _tc_format_output(gp)
    return jnp.transpose(out_t, (2, 0, 1))
